# baseline (speedup 1.0000x reference)
import jax
import jax.numpy as jnp
from jax.experimental import pallas as pl
from jax.experimental.pallas import tpu as pltpu

_MAX_STEPS = 3
_N_EXP = 8
_D_MODEL = 768
_D_FF = 3072
_N_TOK = 2048

_ROW_BLK = 256
_N_ROW = _N_TOK // _ROW_BLK


def _route(pooled, wr_ref, keys_ref, chosen_ref, gate_ref):
    rv = jnp.tanh(jnp.dot(pooled, wr_ref[...], preferred_element_type=jnp.float32))
    scores = jax.lax.dot_general(
        rv, keys_ref[...], (((1,), (1,)), ((), ())),
        preferred_element_type=jnp.float32)
    m = jnp.max(scores)
    idx = jax.lax.broadcasted_iota(jnp.int32, (1, _N_EXP), 1)
    chosen = jnp.min(jnp.where(scores == m, idx, _N_EXP))
    e = jnp.exp(scores - m)
    gate = jnp.sum(jnp.where(idx == chosen, e, 0.0)) / jnp.sum(e)
    chosen_ref[0, 0] = chosen
    gate_ref[0, 0] = gate


def _mega_body(enc_ref, wc_ref, ws_ref, wr_ref, keys_ref, ew1_ref, ew2_ref,
               state_ref, w1_v, w2a_v, w2b_v, hpool_ref, chosen_ref, gate_ref,
               done_ref, w1_sem, w2_sem):
    ones_row = jnp.ones((1, _ROW_BLK), jnp.float32)

    done_ref[0, 0] = 0
    hpool_ref[...] = jnp.zeros_like(hpool_ref)

    def pre_tile(r):
        rows = pl.ds(r * _ROW_BLK, _ROW_BLK)
        t = jnp.tanh(jnp.dot(enc_ref[rows, :], wc_ref[...],
                             preferred_element_type=jnp.float32))
        hpool_ref[0:1, 0:_D_MODEL] += jnp.dot(ones_row, t,
                                              preferred_element_type=jnp.float32)
        x = jnp.dot(t, ws_ref[...], preferred_element_type=jnp.float32)
        state_ref[rows, :] = x

    jax.lax.fori_loop(0, _N_ROW - 1, lambda r, c: (pre_tile(r), c)[1], 0)
    rows_l = pl.ds((_N_ROW - 1) * _ROW_BLK, _ROW_BLK)
    t_l = jnp.tanh(jnp.dot(enc_ref[rows_l, :], wc_ref[...],
                           preferred_element_type=jnp.float32))
    tpool = hpool_ref[0:1, 0:_D_MODEL] + jnp.dot(
        ones_row, t_l, preferred_element_type=jnp.float32)
    pooled0 = jnp.dot(tpool, ws_ref[...],
                      preferred_element_type=jnp.float32) * (1.0 / _N_TOK)
    _route(pooled0, wr_ref, keys_ref, chosen_ref, gate_ref)
    c0 = chosen_ref[0, 0]
    pltpu.make_async_copy(ew1_ref.at[c0], w1_v, w1_sem).start()
    pltpu.make_async_copy(ew2_ref.at[c0], w2a_v, w2_sem).start()
    state_ref[rows_l, :] = jnp.dot(t_l, ws_ref[...],
                                   preferred_element_type=jnp.float32)

    def emit_step(w2_cur, w2_nxt, last):
        c = chosen_ref[0, 0]
        g = gate_ref[0, 0]
        pltpu.make_async_copy(ew1_ref.at[c], w1_v, w1_sem).wait()
        pltpu.make_async_copy(ew2_ref.at[c], w2_cur, w2_sem).wait()
        if not last:
            hpool_ref[...] = jnp.zeros_like(hpool_ref)

        def ffn_tile(r, accumulate_pool):
            rows = pl.ds(r * _ROW_BLK, _ROW_BLK)
            h = jax.nn.gelu(jnp.dot(state_ref[rows, :], w1_v[...],
                                    preferred_element_type=jnp.float32))
            if accumulate_pool:
                hpool_ref[...] += jnp.dot(ones_row, h,
                                          preferred_element_type=jnp.float32)
            o = jnp.dot(h, w2_cur[...], preferred_element_type=jnp.float32) * g
            state_ref[rows, :] = o
            return h

        jax.lax.fori_loop(
            0, _N_ROW - 1, lambda r, cy: (ffn_tile(r, not last), cy)[1], 0)

        if last:
            ffn_tile(_N_ROW - 1, False)
            return

        rows = pl.ds((_N_ROW - 1) * _ROW_BLK, _ROW_BLK)
        h_l = jax.nn.gelu(jnp.dot(state_ref[rows, :], w1_v[...],
                                  preferred_element_type=jnp.float32))
        hp = hpool_ref[...] + jnp.dot(ones_row, h_l,
                                      preferred_element_type=jnp.float32)
        pooled = jnp.dot(hp, w2_cur[...],
                         preferred_element_type=jnp.float32) * (g / _N_TOK)
        _route(pooled, wr_ref, keys_ref, chosen_ref, gate_ref)

        @pl.when(c == 0)
        def _():
            done_ref[0, 0] = 1

        @pl.when(c != 0)
        def _():
            cn = chosen_ref[0, 0]
            pltpu.make_async_copy(ew1_ref.at[cn], w1_v, w1_sem).start()
            pltpu.make_async_copy(ew2_ref.at[cn], w2_nxt, w2_sem).start()

        o_l = jnp.dot(h_l, w2_cur[...], preferred_element_type=jnp.float32) * g
        state_ref[rows, :] = o_l

    emit_step(w2a_v, w2b_v, last=False)

    @pl.when(done_ref[0, 0] == 0)
    def _():
        emit_step(w2b_v, w2a_v, last=False)

    @pl.when(done_ref[0, 0] == 0)
    def _():
        emit_step(w2a_v, w2b_v, last=True)


def kernel(encodings, W_combine, W_router, W_state, expert_keys, expert_W1, expert_W2):
    return pl.pallas_call(
        _mega_body,
        in_specs=[
            pl.BlockSpec(memory_space=pltpu.MemorySpace.VMEM),
            pl.BlockSpec(memory_space=pltpu.MemorySpace.VMEM),
            pl.BlockSpec(memory_space=pltpu.MemorySpace.VMEM),
            pl.BlockSpec(memory_space=pltpu.MemorySpace.VMEM),
            pl.BlockSpec(memory_space=pltpu.MemorySpace.VMEM),
            pl.BlockSpec(memory_space=pltpu.MemorySpace.HBM),
            pl.BlockSpec(memory_space=pltpu.MemorySpace.HBM),
        ],
        out_specs=pl.BlockSpec(memory_space=pltpu.MemorySpace.VMEM),
        out_shape=jax.ShapeDtypeStruct((_N_TOK, _D_MODEL), jnp.float32),
        scratch_shapes=[
            pltpu.VMEM((_D_MODEL, _D_FF), jnp.float32),
            pltpu.VMEM((_D_FF, _D_MODEL), jnp.float32),
            pltpu.VMEM((_D_FF, _D_MODEL), jnp.float32),
            pltpu.VMEM((1, _D_FF), jnp.float32),
            pltpu.SMEM((1, 1), jnp.int32),
            pltpu.SMEM((1, 1), jnp.float32),
            pltpu.SMEM((1, 1), jnp.int32),
            pltpu.SemaphoreType.DMA,
            pltpu.SemaphoreType.DMA,
        ],
    )(encodings, W_combine, W_state, W_router, expert_keys, expert_W1, expert_W2)


# speedup vs baseline: 1.0328x; 1.0328x over previous
import jax
import jax.numpy as jnp
from jax.experimental import pallas as pl
from jax.experimental.pallas import tpu as pltpu

_MAX_STEPS = 3
_N_EXP = 8
_D_MODEL = 768
_D_FF = 3072
_N_TOK = 2048

_ROW_BLK = 256
_N_ROW = _N_TOK // _ROW_BLK
_FF_H = _D_FF // 2


def _route(psum_ref, wr_ref, keys_ref, chosen_ref, gate_ref):
    pooled = psum_ref[...] * (1.0 / _N_TOK)
    rv = jnp.tanh(jnp.dot(pooled, wr_ref[...], preferred_element_type=jnp.float32))
    scores = jax.lax.dot_general(
        rv, keys_ref[...], (((1,), (1,)), ((), ())),
        preferred_element_type=jnp.float32)
    m = jnp.max(scores)
    idx = jax.lax.broadcasted_iota(jnp.int32, (1, _N_EXP), 1)
    chosen = jnp.min(jnp.where(scores == m, idx, _N_EXP))
    e = jnp.exp(scores - m)
    gate = jnp.sum(jnp.where(idx == chosen, e, 0.0)) / jnp.sum(e)
    chosen_ref[0, 0] = chosen
    gate_ref[0, 0] = gate


def _mega_body(enc_ref, wc_ref, ws_ref, wr_ref, keys_ref, ew1_ref, ew2_ref,
               state_ref, w1_v, w2_v, psum_ref, chosen_ref, gate_ref, done_ref,
               w1a_sem, w1b_sem, w2_sem):

    psum_ref[...] = jnp.zeros_like(psum_ref)
    done_ref[0, 0] = 0

    def pre_tile(r, carry):
        rows = pl.ds(r * _ROW_BLK, _ROW_BLK)
        t = jnp.tanh(jnp.dot(enc_ref[rows, :], wc_ref[...],
                             preferred_element_type=jnp.float32))
        x = jnp.dot(t, ws_ref[...], preferred_element_type=jnp.float32)
        state_ref[rows, :] = x
        psum_ref[...] += jnp.sum(x, axis=0, keepdims=True)
        return carry

    jax.lax.fori_loop(0, _N_ROW, pre_tile, 0)
    _route(psum_ref, wr_ref, keys_ref, chosen_ref, gate_ref)

    def emit_step():
        c = chosen_ref[0, 0]
        g = gate_ref[0, 0]
        w1a = pltpu.make_async_copy(
            ew1_ref.at[c, :, pl.ds(0, _FF_H)], w1_v.at[:, pl.ds(0, _FF_H)], w1a_sem)
        w1b = pltpu.make_async_copy(
            ew1_ref.at[c, :, pl.ds(_FF_H, _FF_H)], w1_v.at[:, pl.ds(_FF_H, _FF_H)],
            w1b_sem)
        w2_copy = pltpu.make_async_copy(ew2_ref.at[c], w2_v, w2_sem)
        w1a.start()
        w1b.start()
        w2_copy.start()
        psum_ref[...] = jnp.zeros_like(psum_ref)

        rows0 = pl.ds(0, _ROW_BLK)
        x0 = state_ref[rows0, :]
        w1a.wait()
        h0a = jax.nn.gelu(jnp.dot(x0, w1_v[:, pl.ds(0, _FF_H)],
                                  preferred_element_type=jnp.float32))
        w1b.wait()
        h0b = jax.nn.gelu(jnp.dot(x0, w1_v[:, pl.ds(_FF_H, _FF_H)],
                                  preferred_element_type=jnp.float32))
        w2_copy.wait()
        o0 = (jnp.dot(h0a, w2_v[pl.ds(0, _FF_H), :],
                      preferred_element_type=jnp.float32) +
              jnp.dot(h0b, w2_v[pl.ds(_FF_H, _FF_H), :],
                      preferred_element_type=jnp.float32)) * g
        state_ref[rows0, :] = o0
        psum_ref[...] += jnp.sum(o0, axis=0, keepdims=True)

        def ffn_tile(r, carry):
            rows = pl.ds(r * _ROW_BLK, _ROW_BLK)
            h = jax.nn.gelu(jnp.dot(state_ref[rows, :], w1_v[...],
                                    preferred_element_type=jnp.float32))
            o = jnp.dot(h, w2_v[...], preferred_element_type=jnp.float32) * g
            state_ref[rows, :] = o
            psum_ref[...] += jnp.sum(o, axis=0, keepdims=True)
            return carry

        jax.lax.fori_loop(1, _N_ROW, ffn_tile, 0)

        @pl.when(c == 0)
        def _():
            done_ref[0, 0] = 1

        _route(psum_ref, wr_ref, keys_ref, chosen_ref, gate_ref)

    emit_step()
    for _ in range(_MAX_STEPS - 1):
        @pl.when(done_ref[0, 0] == 0)
        def _():
            emit_step()


def kernel(encodings, W_combine, W_router, W_state, expert_keys, expert_W1, expert_W2):
    return pl.pallas_call(
        _mega_body,
        in_specs=[
            pl.BlockSpec(memory_space=pltpu.MemorySpace.VMEM),
            pl.BlockSpec(memory_space=pltpu.MemorySpace.VMEM),
            pl.BlockSpec(memory_space=pltpu.MemorySpace.VMEM),
            pl.BlockSpec(memory_space=pltpu.MemorySpace.VMEM),
            pl.BlockSpec(memory_space=pltpu.MemorySpace.VMEM),
            pl.BlockSpec(memory_space=pltpu.MemorySpace.HBM),
            pl.BlockSpec(memory_space=pltpu.MemorySpace.HBM),
        ],
        out_specs=pl.BlockSpec(memory_space=pltpu.MemorySpace.VMEM),
        out_shape=jax.ShapeDtypeStruct((_N_TOK, _D_MODEL), jnp.float32),
        scratch_shapes=[
            pltpu.VMEM((_D_MODEL, _D_FF), jnp.float32),
            pltpu.VMEM((_D_FF, _D_MODEL), jnp.float32),
            pltpu.VMEM((1, _D_MODEL), jnp.float32),
            pltpu.SMEM((1, 1), jnp.int32),
            pltpu.SMEM((1, 1), jnp.float32),
            pltpu.SMEM((1, 1), jnp.int32),
            pltpu.SemaphoreType.DMA,
            pltpu.SemaphoreType.DMA,
            pltpu.SemaphoreType.DMA,
        ],
    )(encodings, W_combine, W_state, W_router, expert_keys, expert_W1, expert_W2)
